# initial kernel scaffold (unmeasured)
import jax
import jax.numpy as jnp
from jax import lax
from jax.experimental import pallas as pl
from jax.experimental.pallas import tpu as pltpu


def kernel(
    x,
):
    def body(*refs):
        pass

    out_shape = jax.ShapeDtypeStruct(..., jnp.float32)
    return pl.pallas_call(body, out_shape=out_shape)(...)


# baseline (device time: 11766 ns/iter reference)
import jax
import jax.numpy as jnp
from jax import lax
from jax.experimental import pallas as pl
from jax.experimental.pallas import tpu as pltpu

N_Z = 4
CHUNK = 256


def kernel(x):
    _, m, n = x.shape
    assert (m, n) == (256, N_Z * CHUNK)

    def body(x_ref, out_ref, send_ref, recv_ref, send_sems, recv_sems):
        my_x = lax.axis_index("x")
        my_y = lax.axis_index("y")
        my_z = lax.axis_index("z")

        barrier_sem = pltpu.get_barrier_semaphore()
        for r in range(1, N_Z):
            peer = lax.rem(my_z + r, N_Z)
            pl.semaphore_signal(
                barrier_sem, inc=1,
                device_id=(my_x, my_y, peer),
                device_id_type=pl.DeviceIdType.MESH,
            )
        pl.semaphore_wait(barrier_sem, N_Z - 1)

        rdmas = []
        for r in range(1, N_Z):
            dest = lax.rem(my_z + r, N_Z)
            send_ref[r - 1] = x_ref[0, :, pl.ds(dest * CHUNK, CHUNK)].astype(
                jnp.bfloat16
            )
            rdma = pltpu.make_async_remote_copy(
                src_ref=send_ref.at[r - 1],
                dst_ref=recv_ref.at[r - 1],
                send_sem=send_sems.at[r - 1],
                recv_sem=recv_sems.at[r - 1],
                device_id=(my_x, my_y, dest),
                device_id_type=pl.DeviceIdType.MESH,
            )
            rdma.start()
            rdmas.append(rdma)

        acc = x_ref[0, :, pl.ds(my_z * CHUNK, CHUNK)]
        for r in range(1, N_Z):
            rdmas[r - 1].wait_recv()
            acc = acc + recv_ref[r - 1].astype(jnp.float32)
        out_ref[...] = acc

        for rdma in rdmas:
            rdma.wait_send()

    return pl.pallas_call(
        body,
        out_shape=jax.ShapeDtypeStruct((m, CHUNK), jnp.float32),
        in_specs=[pl.BlockSpec(memory_space=pltpu.VMEM)],
        out_specs=pl.BlockSpec(memory_space=pltpu.VMEM),
        scratch_shapes=[
            pltpu.VMEM((N_Z - 1, m, CHUNK), jnp.bfloat16),
            pltpu.VMEM((N_Z - 1, m, CHUNK), jnp.bfloat16),
            pltpu.SemaphoreType.DMA((N_Z - 1,)),
            pltpu.SemaphoreType.DMA((N_Z - 1,)),
        ],
        compiler_params=pltpu.CompilerParams(collective_id=0),
    )(x)
